# baseline (speedup 1.0000x reference)
import jax
import jax.numpy as jnp
from jax.experimental import pallas as pl
from jax.experimental.pallas import tpu as pltpu
from jax.experimental.pallas import tpu_sc as plsc

_B, _T, _D, _E, _DF = 64, 256, 384, 4, 1536
_N = _B * _T
_EP = 128
_TMA = 1024
_GA = _N // _TMA
_TMC = 256
_L = _N + 6 * _TMC
_NTC = _L // _TMC
_PAIR_E1 = (0, 0, 0, 1, 1, 2)
_PAIR_E2 = (1, 2, 3, 2, 3, 3)
_W = 128
_NCH = 2
_NH = _N // _NCH
_GAH = _NH // _TMA
_LH = _NH + 6 * _TMC
_NTCH = _LH // _TMC


def _lane_major(col, dmask):
    rows = []
    for s in range(_TMA // 128):
        v = col[s * 128:(s + 1) * 128, :]
        rows.append(jnp.sum(jnp.broadcast_to(v, (128, 128)) * dmask,
                            axis=0, keepdims=True))
    return jnp.concatenate(rows, axis=0)


def _gate_kernel(x_ref, gw_ref, gb_ref, ltri_ref,
                 meta_ref, grk_ref, pid_ref, cnt_ref, bal_ref, run_ref):
    i = pl.program_id(0)
    xt = x_ref[...]
    scores = jnp.dot(xt, gw_ref[...], preferred_element_type=jnp.float32)
    scores = scores + gb_ref[...]
    scores = jnp.nan_to_num(scores, nan=0.0)
    m = jnp.max(scores, axis=1, keepdims=True)
    ex = jnp.exp(scores - m)
    probs = ex / jnp.sum(ex, axis=1, keepdims=True)

    @pl.when(i == 0)
    def _init():
        bal_ref[...] = jnp.zeros_like(bal_ref)
        run_ref[...] = jnp.zeros_like(run_ref)
    psum = jnp.sum(probs, axis=0, keepdims=True)
    bal_ref[...] += jnp.broadcast_to(psum, bal_ref.shape)

    lane = jax.lax.broadcasted_iota(jnp.int32, probs.shape, 1)
    m1 = jnp.max(probs, axis=1, keepdims=True)
    a1 = jnp.min(jnp.where(probs == m1, lane, _EP), axis=1, keepdims=True)
    p2 = jnp.where(lane == a1, -jnp.inf, probs)
    m2 = jnp.max(p2, axis=1, keepdims=True)
    a2 = jnp.min(jnp.where(p2 == m2, lane, _EP), axis=1, keepdims=True)
    sel = (lane == a1) | (lane == a2)
    masked = jnp.where(sel, probs, 0.0)
    wgt = masked / (jnp.sum(masked, axis=1, keepdims=True) + 1e-9)
    meta_ref[...] = wgt * (lane < _E)

    emin = jnp.minimum(a1, a2)
    emax = jnp.maximum(a1, a2)
    pid = emax + jnp.where(emin == 0, -1, emin)

    onehot = (lane == pid)
    cnts = jnp.dot(ltri_ref[...], onehot.astype(jnp.bfloat16),
                   preferred_element_type=jnp.float32)
    rank = jnp.sum(jnp.where(onehot, cnts, 0.0), axis=1, keepdims=True)
    runrow = jnp.broadcast_to(run_ref[0:1, :], onehot.shape)
    grank = rank + jnp.sum(jnp.where(onehot, runrow, 0.0), axis=1,
                           keepdims=True)
    run_ref[...] += jnp.broadcast_to(
        jnp.sum(onehot.astype(jnp.float32), axis=0, keepdims=True),
        run_ref.shape)

    sub = jax.lax.broadcasted_iota(jnp.int32, (128, 128), 0)
    lan = jax.lax.broadcasted_iota(jnp.int32, (128, 128), 1)
    dmask = (sub == lan).astype(jnp.float32)
    grk_ref[...] = _lane_major(grank, dmask)
    pid_ref[...] = _lane_major(pid.astype(jnp.float32), dmask)

    @pl.when(i == pl.num_programs(0) - 1)
    def _fin():
        cnt_ref[...] = run_ref[...]


def _dest_kernel(off_ref, grk_ref, pid_ref, dest_ref):
    grk = grk_ref[...]
    pidf = pid_ref[...]
    acc = grk
    for p in range(6):
        acc = acc + jnp.where(pidf == float(p),
                              off_ref[p].astype(jnp.float32), 0.0)
    dest_ref[...] = acc.astype(jnp.int32)


def _ffn_kernel(e1s_ref, e2s_ref, xs_ref, ws_ref,
                w1_ref, b1_ref, w2_ref, b2_ref, ys_ref):
    tt = pl.program_id(0)
    e1 = e1s_ref[tt]
    e2 = e2s_ref[tt]
    xt = xs_ref[...]
    ws = ws_ref[...]
    lane = jax.lax.broadcasted_iota(jnp.int32, ws.shape, 1)
    wa = jnp.sum(jnp.where(lane == e1, ws, 0.0), axis=1, keepdims=True)
    wb = jnp.sum(jnp.where(lane == e2, ws, 0.0), axis=1, keepdims=True)

    acc = jnp.zeros((_TMC, _D), dtype=jnp.float32)
    for e, w in ((e1, wa), (e2, wb)):
        t = jnp.dot(xt, w1_ref[e], preferred_element_type=jnp.float32)
        t = t + b1_ref[e]
        t = 0.5 * t * (1.0 + jax.lax.erf(t * 0.7071067811865476))
        y = jnp.dot(t, w2_ref[e], preferred_element_type=jnp.float32)
        acc = acc + w * (y + b2_ref[e])
    ys_ref[...] = acc


def _vector_mesh():
    return plsc.VectorSubcoreMesh(core_axis_name="core",
                                  subcore_axis_name="subcore")


def _sc_scatter_rows(src, idx, n_slots, blk_off):
    ncol = src.shape[1]

    @pl.kernel(out_type=jax.ShapeDtypeStruct((n_slots, ncol), src.dtype),
               mesh=_vector_mesh(), scratch_types=[])
    def skern(s_hbm, i_hbm, o_hbm):
        def body(s_vmem, i_vmem):
            pltpu.sync_copy(s_vmem, o_hbm.at[i_vmem.at[0]])

        pltpu.emit_pipeline(
            body,
            grid=(_NH // _W,),
            in_specs=[pl.BlockSpec((_W, ncol), lambda i: (i + blk_off, 0)),
                      pl.BlockSpec((1, _W), lambda i: (0, i))],
            out_specs=[],
            core_axis_name=("core", "subcore"),
            dimension_semantics=(pltpu.PARALLEL,),
        )(s_hbm, i_hbm)

    return skern(src, idx)


def _sc_gather2(ys_list, idx_list):
    @pl.kernel(out_type=jax.ShapeDtypeStruct((_N, _D), jnp.float32),
               mesh=_vector_mesh(), scratch_types=[])
    def gkern(*refs):
        y_hbms = refs[:_NCH]
        i_hbms = refs[_NCH:2 * _NCH]
        o_hbm = refs[2 * _NCH]
        for h in range(_NCH):
            y_hbm = y_hbms[h]
            out_off = h * (_NH // _W)

            def body(i_vmem, o_vmem, y_hbm=y_hbm):
                pltpu.sync_copy(y_hbm.at[i_vmem.at[0]], o_vmem)

            pltpu.emit_pipeline(
                body,
                grid=(_NH // _W,),
                in_specs=[pl.BlockSpec((1, _W), lambda i: (0, i))],
                out_specs=[pl.BlockSpec((_W, _D),
                                        lambda i, o=out_off: (i + o, 0))],
                core_axis_name=("core", "subcore"),
                dimension_semantics=(pltpu.PARALLEL,),
            )(i_hbms[h], o_hbm)

    return gkern(*ys_list, *idx_list)


def kernel(x, gate_w, gate_b, w1, b1, w2, b2):
    x2 = x.reshape(_N, _D)
    gw_p = jnp.zeros((_D, _EP), jnp.float32).at[:, :_E].set(gate_w)
    gb_p = jnp.full((1, _EP), -jnp.inf, jnp.float32).at[0, :_E].set(gate_b)
    ltri = jnp.tril(jnp.ones((_TMA, _TMA), jnp.bfloat16), -1)
    b1r = b1.reshape(_E, 1, _DF)
    b2r = b2.reshape(_E, 1, _D)

    ys_list, dest_list, bal_list = [], [], []
    for h in range(_NCH):
        toff = h * (_NH // _TMA)
        meta, grk, pidm, cnt, bal = pl.pallas_call(
            _gate_kernel,
            grid=(_GAH,),
            in_specs=[
                pl.BlockSpec((_TMA, _D), lambda i, t=toff: (i + t, 0)),
                pl.BlockSpec((_D, _EP), lambda i: (0, 0)),
                pl.BlockSpec((1, _EP), lambda i: (0, 0)),
                pl.BlockSpec((_TMA, _TMA), lambda i: (0, 0)),
            ],
            out_specs=[
                pl.BlockSpec((_TMA, _EP), lambda i: (i, 0)),
                pl.BlockSpec((8, _EP), lambda i: (i, 0)),
                pl.BlockSpec((8, _EP), lambda i: (i, 0)),
                pl.BlockSpec((8, _EP), lambda i: (0, 0)),
                pl.BlockSpec((8, _EP), lambda i: (0, 0)),
            ],
            out_shape=[
                jax.ShapeDtypeStruct((_NH, _EP), jnp.float32),
                jax.ShapeDtypeStruct((_GAH * 8, _EP), jnp.float32),
                jax.ShapeDtypeStruct((_GAH * 8, _EP), jnp.float32),
                jax.ShapeDtypeStruct((8, _EP), jnp.float32),
                jax.ShapeDtypeStruct((8, _EP), jnp.float32),
            ],
            scratch_shapes=[pltpu.VMEM((8, _EP), jnp.float32)],
        )(x2, gw_p, gb_p, ltri)
        bal_list.append(bal)

        cnt6 = cnt[0, :6]
        rup = jnp.ceil(cnt6 / _TMC) * _TMC
        ends = jnp.cumsum(rup)
        off6 = ends - rup
        off8 = jnp.zeros((8,), jnp.int32).at[:6].set(off6.astype(jnp.int32))
        tt0 = jnp.arange(_NTCH, dtype=jnp.float32) * _TMC
        gid = jnp.sum(tt0[:, None] >= ends[None, :], axis=1).astype(jnp.int32)
        gid = jnp.minimum(gid, 5)
        e1s = jnp.asarray(_PAIR_E1, jnp.int32)[gid]
        e2s = jnp.asarray(_PAIR_E2, jnp.int32)[gid]

        destb = pl.pallas_call(
            _dest_kernel,
            grid_spec=pltpu.PrefetchScalarGridSpec(
                num_scalar_prefetch=1,
                grid=(1,),
                in_specs=[
                    pl.BlockSpec((_GAH * 8, _EP), lambda i, s: (0, 0)),
                    pl.BlockSpec((_GAH * 8, _EP), lambda i, s: (0, 0)),
                ],
                out_specs=pl.BlockSpec((_GAH * 8, _EP), lambda i, s: (0, 0)),
            ),
            out_shape=jax.ShapeDtypeStruct((_GAH * 8, _EP), jnp.int32),
        )(off8, grk, pidm)
        dest = destb.reshape(1, _NH)
        dest_list.append(dest)

        xs = _sc_scatter_rows(x2, dest, _LH, h * (_NH // _W))
        ws = _sc_scatter_rows(meta, dest, _LH, 0)

        ys = pl.pallas_call(
            _ffn_kernel,
            grid_spec=pltpu.PrefetchScalarGridSpec(
                num_scalar_prefetch=2,
                grid=(_NTCH,),
                in_specs=[
                    pl.BlockSpec((_TMC, _D), lambda i, s1, s2: (i, 0)),
                    pl.BlockSpec((_TMC, _EP), lambda i, s1, s2: (i, 0)),
                    pl.BlockSpec((_E, _D, _DF), lambda i, s1, s2: (0, 0, 0)),
                    pl.BlockSpec((_E, 1, _DF), lambda i, s1, s2: (0, 0, 0)),
                    pl.BlockSpec((_E, _DF, _D), lambda i, s1, s2: (0, 0, 0)),
                    pl.BlockSpec((_E, 1, _D), lambda i, s1, s2: (0, 0, 0)),
                ],
                out_specs=pl.BlockSpec((_TMC, _D), lambda i, s1, s2: (i, 0)),
            ),
            out_shape=jax.ShapeDtypeStruct((_LH, _D), jnp.float32),
        )(e1s, e2s, xs, ws, w1, b1r, w2, b2r)
        ys_list.append(ys)

    out = _sc_gather2(ys_list, dest_list)

    bal = bal_list[0] + bal_list[1]
    bl = (jnp.sum((bal[0, :_E] / _N) ** 2)) * _E
    bal_loss = jnp.clip(bl, 0.0, 5.0)
    return out.reshape(_B, _T, _D), bal_loss


# speedup vs baseline: 1.0306x; 1.0306x over previous
import jax
import jax.numpy as jnp
from jax.experimental import pallas as pl
from jax.experimental.pallas import tpu as pltpu
from jax.experimental.pallas import tpu_sc as plsc

_B, _T, _D, _E, _DF = 64, 256, 384, 4, 1536
_N = _B * _T
_EP = 128
_TMA = 1024
_GA = _N // _TMA
_TMC = 256
_L = _N + 6 * _TMC
_NTC = _L // _TMC
_PAIR_E1 = (0, 0, 0, 1, 1, 2)
_PAIR_E2 = (1, 2, 3, 2, 3, 3)
_W = 128
_RS2 = 0.7071067811865476


def _lane_major(col, dmask):
    rows = []
    for s in range(_TMA // 128):
        v = col[s * 128:(s + 1) * 128, :]
        rows.append(jnp.sum(jnp.broadcast_to(v, (128, 128)) * dmask,
                            axis=0, keepdims=True))
    return jnp.concatenate(rows, axis=0)


def _gate_kernel(x_ref, gw_ref, gb_ref, ltri_ref,
                 meta_ref, grk_ref, pid_ref, cnt_ref, bal_ref, run_ref):
    i = pl.program_id(0)
    xt = x_ref[...]
    scores = jnp.dot(xt, gw_ref[...], preferred_element_type=jnp.float32)
    scores = scores + gb_ref[...]
    scores = jnp.nan_to_num(scores, nan=0.0)
    m = jnp.max(scores, axis=1, keepdims=True)
    ex = jnp.exp(scores - m)
    probs = ex / jnp.sum(ex, axis=1, keepdims=True)

    @pl.when(i == 0)
    def _init():
        bal_ref[...] = jnp.zeros_like(bal_ref)
        run_ref[...] = jnp.zeros_like(run_ref)
    psum = jnp.sum(probs, axis=0, keepdims=True)
    bal_ref[...] += jnp.broadcast_to(psum, bal_ref.shape)

    lane = jax.lax.broadcasted_iota(jnp.int32, probs.shape, 1)
    m1 = jnp.max(probs, axis=1, keepdims=True)
    a1 = jnp.min(jnp.where(probs == m1, lane, _EP), axis=1, keepdims=True)
    p2 = jnp.where(lane == a1, -jnp.inf, probs)
    m2 = jnp.max(p2, axis=1, keepdims=True)
    a2 = jnp.min(jnp.where(p2 == m2, lane, _EP), axis=1, keepdims=True)
    sel = (lane == a1) | (lane == a2)
    masked = jnp.where(sel, probs, 0.0)
    wgt = masked / (jnp.sum(masked, axis=1, keepdims=True) + 1e-9)
    meta_ref[...] = wgt * (lane < _E)

    emin = jnp.minimum(a1, a2)
    emax = jnp.maximum(a1, a2)
    pid = emax + jnp.where(emin == 0, -1, emin)

    onehot = (lane == pid)
    cnts = jnp.dot(ltri_ref[...], onehot.astype(jnp.bfloat16),
                   preferred_element_type=jnp.float32)
    rank = jnp.sum(jnp.where(onehot, cnts, 0.0), axis=1, keepdims=True)
    runrow = jnp.broadcast_to(run_ref[0:1, :], onehot.shape)
    grank = rank + jnp.sum(jnp.where(onehot, runrow, 0.0), axis=1,
                           keepdims=True)
    run_ref[...] += jnp.broadcast_to(
        jnp.sum(onehot.astype(jnp.float32), axis=0, keepdims=True),
        run_ref.shape)

    sub = jax.lax.broadcasted_iota(jnp.int32, (128, 128), 0)
    lan = jax.lax.broadcasted_iota(jnp.int32, (128, 128), 1)
    dmask = (sub == lan).astype(jnp.float32)
    grk_ref[...] = _lane_major(grank, dmask)
    pid_ref[...] = _lane_major(pid.astype(jnp.float32), dmask)

    @pl.when(i == _GA - 1)
    def _fin():
        cnt_ref[...] = run_ref[...]


def _dest_kernel(cnt_ref, grk_ref, pid_ref, bal_ref,
                 dest_ref, e1v_ref, e2v_ref, bl_ref):
    sub = jax.lax.broadcasted_iota(jnp.int32, (128, 128), 0)
    lan = jax.lax.broadcasted_iota(jnp.int32, (128, 128), 1)
    dmask = (sub == lan).astype(jnp.float32)

    lane1 = jax.lax.broadcasted_iota(jnp.int32, (1, _EP), 1)
    cnt = jnp.where(lane1 < 6, cnt_ref[0:1, :], 0.0)
    rup = jnp.ceil(cnt / _TMC) * _TMC
    utri = (sub <= lan).astype(jnp.float32)
    ends = jnp.dot(rup, utri, preferred_element_type=jnp.float32)
    offs = ends - rup

    grk = grk_ref[...]
    pidf = pid_ref[...]
    acc = grk
    for p in range(6):
        acc = acc + jnp.where(pidf == float(p),
                              jnp.broadcast_to(offs[:, p:p + 1], grk.shape),
                              0.0)
    dest_ref[...] = acc.astype(jnp.int32)

    ends_t = jnp.sum(jnp.broadcast_to(jnp.where(lane1 < 6, ends, 3.0e7),
                                      (128, 128)) * dmask,
                     axis=1, keepdims=True)
    ttv = (lan * _TMC).astype(jnp.float32)
    gid = jnp.sum((ttv >= ends_t).astype(jnp.float32), axis=0, keepdims=True)
    gid = jnp.minimum(gid, 5.0)
    e1 = jnp.zeros_like(gid)
    e2 = jnp.zeros_like(gid)
    for g in range(6):
        e1 = e1 + jnp.where(gid == float(g), float(_PAIR_E1[g]), 0.0)
        e2 = e2 + jnp.where(gid == float(g), float(_PAIR_E2[g]), 0.0)
    e1v_ref[...] = jnp.broadcast_to(e1, (8, _EP)).astype(jnp.int32)
    e2v_ref[...] = jnp.broadcast_to(e2, (8, _EP)).astype(jnp.int32)

    bsum = jnp.where(lane1 < _E, bal_ref[0:1, :], 0.0) * (1.0 / _N)
    bl = jnp.sum(bsum * bsum) * _E
    bl = jnp.clip(bl, 0.0, 5.0)
    bl_ref[...] = jnp.broadcast_to(bl, (8, _EP))


def _ffn_kernel(e1s_ref, e2s_ref, xs_ref, ws_ref,
                w1_ref, b1_ref, w2_ref, b2_ref, ys_ref):
    tt = pl.program_id(0)
    e1 = e1s_ref[0, tt]
    e2 = e2s_ref[0, tt]
    xt = xs_ref[...]
    ws = ws_ref[...]
    lane = jax.lax.broadcasted_iota(jnp.int32, ws.shape, 1)
    wa = jnp.sum(jnp.where(lane == e1, ws, 0.0), axis=1, keepdims=True)
    wb = jnp.sum(jnp.where(lane == e2, ws, 0.0), axis=1, keepdims=True)

    acc = jnp.zeros((_TMC, _D), dtype=jnp.float32)
    for e, w in ((e1, wa), (e2, wb)):
        t = jnp.dot(xt, w1_ref[e], preferred_element_type=jnp.float32)
        t = t + b1_ref[e]
        t = 0.5 * t * (1.0 + jax.lax.erf(t * 0.7071067811865476))
        y = jnp.dot(t, w2_ref[e], preferred_element_type=jnp.float32)
        acc = acc + w * (y + b2_ref[e])
    ys_ref[...] = acc


def _vector_mesh():
    return plsc.VectorSubcoreMesh(core_axis_name="core",
                                  subcore_axis_name="subcore")


def _sc_scatter_rows(src, idx, n_slots):
    ncol = src.shape[1]

    @pl.kernel(out_type=jax.ShapeDtypeStruct((n_slots, ncol), src.dtype),
               mesh=_vector_mesh(), scratch_types=[])
    def skern(s_hbm, i_hbm, o_hbm):
        def body(s_vmem, i_vmem):
            pltpu.sync_copy(s_vmem, o_hbm.at[i_vmem.at[0]])

        pltpu.emit_pipeline(
            body,
            grid=(_N // _W,),
            in_specs=[pl.BlockSpec((_W, ncol), lambda i: (i, 0)),
                      pl.BlockSpec((1, _W), lambda i: (0, i))],
            out_specs=[],
            core_axis_name=("core", "subcore"),
            dimension_semantics=(pltpu.PARALLEL,),
        )(s_hbm, i_hbm)

    return skern(src, idx)


def _sc_gather(ys, idx):
    @pl.kernel(out_type=jax.ShapeDtypeStruct((_N, _D), jnp.float32),
               mesh=_vector_mesh(), scratch_types=[])
    def gkern(y_hbm, i_hbm, o_hbm):
        def body(i_vmem, o_vmem):
            pltpu.sync_copy(y_hbm.at[i_vmem.at[0]], o_vmem)

        pltpu.emit_pipeline(
            body,
            grid=(_N // _W,),
            in_specs=[pl.BlockSpec((1, _W), lambda i: (0, i))],
            out_specs=[pl.BlockSpec((_W, _D), lambda i: (i, 0))],
            core_axis_name=("core", "subcore"),
            dimension_semantics=(pltpu.PARALLEL,),
        )(i_hbm, o_hbm)

    return gkern(ys, idx)


def kernel(x, gate_w, gate_b, w1, b1, w2, b2):
    x2 = x.reshape(_N, _D)
    gw_p = jnp.zeros((_D, _EP), jnp.float32).at[:, :_E].set(gate_w)
    gb_p = jnp.full((1, _EP), -jnp.inf, jnp.float32).at[0, :_E].set(gate_b)
    ltri = jnp.tril(jnp.ones((_TMA, _TMA), jnp.bfloat16), -1)

    meta, grk, pidm, cnt, bal = pl.pallas_call(
        _gate_kernel,
        grid=(_GA,),
        in_specs=[
            pl.BlockSpec((_TMA, _D), lambda i: (i, 0)),
            pl.BlockSpec((_D, _EP), lambda i: (0, 0)),
            pl.BlockSpec((1, _EP), lambda i: (0, 0)),
            pl.BlockSpec((_TMA, _TMA), lambda i: (0, 0)),
        ],
        out_specs=[
            pl.BlockSpec((_TMA, _EP), lambda i: (i, 0)),
            pl.BlockSpec((8, _EP), lambda i: (i, 0)),
            pl.BlockSpec((8, _EP), lambda i: (i, 0)),
            pl.BlockSpec((8, _EP), lambda i: (0, 0)),
            pl.BlockSpec((8, _EP), lambda i: (0, 0)),
        ],
        out_shape=[
            jax.ShapeDtypeStruct((_N, _EP), jnp.float32),
            jax.ShapeDtypeStruct((_GA * 8, _EP), jnp.float32),
            jax.ShapeDtypeStruct((_GA * 8, _EP), jnp.float32),
            jax.ShapeDtypeStruct((8, _EP), jnp.float32),
            jax.ShapeDtypeStruct((8, _EP), jnp.float32),
        ],
        scratch_shapes=[pltpu.VMEM((8, _EP), jnp.float32)],
    )(x2, gw_p, gb_p, ltri)

    destb, e1s, e2s, blv = pl.pallas_call(
        _dest_kernel,
        grid=(1,),
        in_specs=[
            pl.BlockSpec((8, _EP), lambda i: (0, 0)),
            pl.BlockSpec((_GA * 8, _EP), lambda i: (0, 0)),
            pl.BlockSpec((_GA * 8, _EP), lambda i: (0, 0)),
            pl.BlockSpec((8, _EP), lambda i: (0, 0)),
        ],
        out_specs=[
            pl.BlockSpec((_GA * 8, _EP), lambda i: (0, 0)),
            pl.BlockSpec((8, _EP), lambda i: (0, 0)),
            pl.BlockSpec((8, _EP), lambda i: (0, 0)),
            pl.BlockSpec((8, _EP), lambda i: (0, 0)),
        ],
        out_shape=[
            jax.ShapeDtypeStruct((_GA * 8, _EP), jnp.int32),
            jax.ShapeDtypeStruct((8, _EP), jnp.int32),
            jax.ShapeDtypeStruct((8, _EP), jnp.int32),
            jax.ShapeDtypeStruct((8, _EP), jnp.float32),
        ],
    )(cnt, grk, pidm, bal)
    dest = destb.reshape(1, _N)

    xs = _sc_scatter_rows(x2, dest, _L)
    ws = _sc_scatter_rows(meta, dest, _L)

    ys = pl.pallas_call(
        _ffn_kernel,
        grid_spec=pltpu.PrefetchScalarGridSpec(
            num_scalar_prefetch=2,
            grid=(_NTC,),
            in_specs=[
                pl.BlockSpec((_TMC, _D), lambda i, s1, s2: (i, 0)),
                pl.BlockSpec((_TMC, _EP), lambda i, s1, s2: (i, 0)),
                pl.BlockSpec((_E, _D, _DF), lambda i, s1, s2: (0, 0, 0)),
                pl.BlockSpec((_E, 1, _DF), lambda i, s1, s2: (0, 0, 0)),
                pl.BlockSpec((_E, _DF, _D), lambda i, s1, s2: (0, 0, 0)),
                pl.BlockSpec((_E, 1, _D), lambda i, s1, s2: (0, 0, 0)),
            ],
            out_specs=pl.BlockSpec((_TMC, _D), lambda i, s1, s2: (i, 0)),
        ),
        out_shape=jax.ShapeDtypeStruct((_L, _D), jnp.float32),
    )(e1s, e2s, xs, ws, w1, b1.reshape(_E, 1, _DF), w2,
      b2.reshape(_E, 1, _D))

    out = _sc_gather(ys, dest)

    bal_loss = blv[0, 0]
    return out.reshape(_B, _T, _D), bal_loss


# speedup vs baseline: 1.0610x; 1.0295x over previous
import jax
import jax.numpy as jnp
from jax.experimental import pallas as pl
from jax.experimental.pallas import tpu as pltpu
from jax.experimental.pallas import tpu_sc as plsc

_B, _T, _D, _E, _DF = 64, 256, 384, 4, 1536
_N = _B * _T
_EP = 128
_TMA = 1024
_GA = _N // _TMA
_TMC = 512
_L = _N + 6 * _TMC
_NTC = _L // _TMC
_PAIR_E1 = (0, 0, 0, 1, 1, 2)
_PAIR_E2 = (1, 2, 3, 2, 3, 3)
_W = 128
_RS2 = 0.7071067811865476


def _lane_major(col, dmask):
    rows = []
    for s in range(_TMA // 128):
        v = col[s * 128:(s + 1) * 128, :]
        rows.append(jnp.sum(jnp.broadcast_to(v, (128, 128)) * dmask,
                            axis=0, keepdims=True))
    return jnp.concatenate(rows, axis=0)


def _gate_kernel(x_ref, gw_ref, gb_ref, ltri_ref,
                 meta_ref, grk_ref, pid_ref, cnt_ref, bal_ref, run_ref):
    i = pl.program_id(0)
    xt = x_ref[...]
    scores = jnp.dot(xt, gw_ref[...], preferred_element_type=jnp.float32)
    scores = scores + gb_ref[...]
    scores = jnp.nan_to_num(scores, nan=0.0)
    m = jnp.max(scores, axis=1, keepdims=True)
    ex = jnp.exp(scores - m)
    probs = ex / jnp.sum(ex, axis=1, keepdims=True)

    @pl.when(i == 0)
    def _init():
        bal_ref[...] = jnp.zeros_like(bal_ref)
        run_ref[...] = jnp.zeros_like(run_ref)
    psum = jnp.sum(probs, axis=0, keepdims=True)
    bal_ref[...] += jnp.broadcast_to(psum, bal_ref.shape)

    lane = jax.lax.broadcasted_iota(jnp.int32, probs.shape, 1)
    m1 = jnp.max(probs, axis=1, keepdims=True)
    a1 = jnp.min(jnp.where(probs == m1, lane, _EP), axis=1, keepdims=True)
    p2 = jnp.where(lane == a1, -jnp.inf, probs)
    m2 = jnp.max(p2, axis=1, keepdims=True)
    a2 = jnp.min(jnp.where(p2 == m2, lane, _EP), axis=1, keepdims=True)
    sel = (lane == a1) | (lane == a2)
    masked = jnp.where(sel, probs, 0.0)
    wgt = masked / (jnp.sum(masked, axis=1, keepdims=True) + 1e-9)
    meta_ref[...] = wgt * (lane < _E)

    emin = jnp.minimum(a1, a2)
    emax = jnp.maximum(a1, a2)
    pid = emax + jnp.where(emin == 0, -1, emin)

    onehot = (lane == pid)
    cnts = jnp.dot(ltri_ref[...], onehot.astype(jnp.bfloat16),
                   preferred_element_type=jnp.float32)
    rank = jnp.sum(jnp.where(onehot, cnts, 0.0), axis=1, keepdims=True)
    runrow = jnp.broadcast_to(run_ref[0:1, :], onehot.shape)
    grank = rank + jnp.sum(jnp.where(onehot, runrow, 0.0), axis=1,
                           keepdims=True)
    run_ref[...] += jnp.broadcast_to(
        jnp.sum(onehot.astype(jnp.float32), axis=0, keepdims=True),
        run_ref.shape)

    sub = jax.lax.broadcasted_iota(jnp.int32, (128, 128), 0)
    lan = jax.lax.broadcasted_iota(jnp.int32, (128, 128), 1)
    dmask = (sub == lan).astype(jnp.float32)
    grk_ref[...] = _lane_major(grank, dmask)
    pid_ref[...] = _lane_major(pid.astype(jnp.float32), dmask)

    @pl.when(i == _GA - 1)
    def _fin():
        cnt_ref[...] = run_ref[...]


def _dest_kernel(cnt_ref, grk_ref, pid_ref, bal_ref,
                 dest_ref, e1v_ref, e2v_ref, bl_ref):
    sub = jax.lax.broadcasted_iota(jnp.int32, (128, 128), 0)
    lan = jax.lax.broadcasted_iota(jnp.int32, (128, 128), 1)
    dmask = (sub == lan).astype(jnp.float32)

    lane1 = jax.lax.broadcasted_iota(jnp.int32, (1, _EP), 1)
    cnt = jnp.where(lane1 < 6, cnt_ref[0:1, :], 0.0)
    rup = jnp.ceil(cnt / _TMC) * _TMC
    utri = (sub <= lan).astype(jnp.float32)
    ends = jnp.dot(rup, utri, preferred_element_type=jnp.float32)
    offs = ends - rup

    grk = grk_ref[...]
    pidf = pid_ref[...]
    acc = grk
    for p in range(6):
        acc = acc + jnp.where(pidf == float(p),
                              jnp.broadcast_to(offs[:, p:p + 1], grk.shape),
                              0.0)
    dest_ref[...] = acc.astype(jnp.int32)

    ends_t = jnp.sum(jnp.broadcast_to(jnp.where(lane1 < 6, ends, 3.0e7),
                                      (128, 128)) * dmask,
                     axis=1, keepdims=True)
    ttv = (lan * _TMC).astype(jnp.float32)
    gid = jnp.sum((ttv >= ends_t).astype(jnp.float32), axis=0, keepdims=True)
    gid = jnp.minimum(gid, 5.0)
    e1 = jnp.zeros_like(gid)
    e2 = jnp.zeros_like(gid)
    for g in range(6):
        e1 = e1 + jnp.where(gid == float(g), float(_PAIR_E1[g]), 0.0)
        e2 = e2 + jnp.where(gid == float(g), float(_PAIR_E2[g]), 0.0)
    e1v_ref[...] = jnp.broadcast_to(e1, (8, _EP)).astype(jnp.int32)
    e2v_ref[...] = jnp.broadcast_to(e2, (8, _EP)).astype(jnp.int32)

    bsum = jnp.where(lane1 < _E, bal_ref[0:1, :], 0.0) * (1.0 / _N)
    bl = jnp.sum(bsum * bsum) * _E
    bl = jnp.clip(bl, 0.0, 5.0)
    bl_ref[...] = jnp.broadcast_to(bl, (8, _EP))


def _ffn_kernel(e1s_ref, e2s_ref, xs_ref, ws_ref,
                w1_ref, b1_ref, w2_ref, b2_ref, ys_ref):
    tt = pl.program_id(0)
    e1 = e1s_ref[0, tt]
    e2 = e2s_ref[0, tt]
    xt = xs_ref[...]
    ws = ws_ref[...]
    lane = jax.lax.broadcasted_iota(jnp.int32, ws.shape, 1)
    wa = jnp.sum(jnp.where(lane == e1, ws, 0.0), axis=1, keepdims=True)
    wb = jnp.sum(jnp.where(lane == e2, ws, 0.0), axis=1, keepdims=True)

    acc = jnp.zeros((_TMC, _D), dtype=jnp.float32)
    for e, w in ((e1, wa), (e2, wb)):
        t = jnp.dot(xt, w1_ref[e], preferred_element_type=jnp.float32)
        t = t + b1_ref[e]
        t = 0.5 * t * (1.0 + jax.lax.erf(t * 0.7071067811865476))
        y = jnp.dot(t, w2_ref[e], preferred_element_type=jnp.float32)
        acc = acc + w * (y + b2_ref[e])
    ys_ref[...] = acc


def _vector_mesh():
    return plsc.VectorSubcoreMesh(core_axis_name="core",
                                  subcore_axis_name="subcore")


def _sc_scatter_rows(src, idx, n_slots):
    ncol = src.shape[1]

    @pl.kernel(out_type=jax.ShapeDtypeStruct((n_slots, ncol), src.dtype),
               mesh=_vector_mesh(), scratch_types=[])
    def skern(s_hbm, i_hbm, o_hbm):
        def body(s_vmem, i_vmem):
            pltpu.sync_copy(s_vmem, o_hbm.at[i_vmem.at[0]])

        pltpu.emit_pipeline(
            body,
            grid=(_N // _W,),
            in_specs=[pl.BlockSpec((_W, ncol), lambda i: (i, 0)),
                      pl.BlockSpec((1, _W), lambda i: (0, i))],
            out_specs=[],
            core_axis_name=("core", "subcore"),
            dimension_semantics=(pltpu.PARALLEL,),
        )(s_hbm, i_hbm)

    return skern(src, idx)


def _sc_gather(ys, idx):
    @pl.kernel(out_type=jax.ShapeDtypeStruct((_N, _D), jnp.float32),
               mesh=_vector_mesh(), scratch_types=[])
    def gkern(y_hbm, i_hbm, o_hbm):
        def body(i_vmem, o_vmem):
            pltpu.sync_copy(y_hbm.at[i_vmem.at[0]], o_vmem)

        pltpu.emit_pipeline(
            body,
            grid=(_N // _W,),
            in_specs=[pl.BlockSpec((1, _W), lambda i: (0, i))],
            out_specs=[pl.BlockSpec((_W, _D), lambda i: (i, 0))],
            core_axis_name=("core", "subcore"),
            dimension_semantics=(pltpu.PARALLEL,),
        )(i_hbm, o_hbm)

    return gkern(ys, idx)


def kernel(x, gate_w, gate_b, w1, b1, w2, b2):
    x2 = x.reshape(_N, _D)
    gw_p = jnp.zeros((_D, _EP), jnp.float32).at[:, :_E].set(gate_w)
    gb_p = jnp.full((1, _EP), -jnp.inf, jnp.float32).at[0, :_E].set(gate_b)
    ltri = jnp.tril(jnp.ones((_TMA, _TMA), jnp.bfloat16), -1)

    meta, grk, pidm, cnt, bal = pl.pallas_call(
        _gate_kernel,
        grid=(_GA,),
        in_specs=[
            pl.BlockSpec((_TMA, _D), lambda i: (i, 0)),
            pl.BlockSpec((_D, _EP), lambda i: (0, 0)),
            pl.BlockSpec((1, _EP), lambda i: (0, 0)),
            pl.BlockSpec((_TMA, _TMA), lambda i: (0, 0)),
        ],
        out_specs=[
            pl.BlockSpec((_TMA, _EP), lambda i: (i, 0)),
            pl.BlockSpec((8, _EP), lambda i: (i, 0)),
            pl.BlockSpec((8, _EP), lambda i: (i, 0)),
            pl.BlockSpec((8, _EP), lambda i: (0, 0)),
            pl.BlockSpec((8, _EP), lambda i: (0, 0)),
        ],
        out_shape=[
            jax.ShapeDtypeStruct((_N, _EP), jnp.float32),
            jax.ShapeDtypeStruct((_GA * 8, _EP), jnp.float32),
            jax.ShapeDtypeStruct((_GA * 8, _EP), jnp.float32),
            jax.ShapeDtypeStruct((8, _EP), jnp.float32),
            jax.ShapeDtypeStruct((8, _EP), jnp.float32),
        ],
        scratch_shapes=[pltpu.VMEM((8, _EP), jnp.float32)],
    )(x2, gw_p, gb_p, ltri)

    destb, e1s, e2s, blv = pl.pallas_call(
        _dest_kernel,
        grid=(1,),
        in_specs=[
            pl.BlockSpec((8, _EP), lambda i: (0, 0)),
            pl.BlockSpec((_GA * 8, _EP), lambda i: (0, 0)),
            pl.BlockSpec((_GA * 8, _EP), lambda i: (0, 0)),
            pl.BlockSpec((8, _EP), lambda i: (0, 0)),
        ],
        out_specs=[
            pl.BlockSpec((_GA * 8, _EP), lambda i: (0, 0)),
            pl.BlockSpec((8, _EP), lambda i: (0, 0)),
            pl.BlockSpec((8, _EP), lambda i: (0, 0)),
            pl.BlockSpec((8, _EP), lambda i: (0, 0)),
        ],
        out_shape=[
            jax.ShapeDtypeStruct((_GA * 8, _EP), jnp.int32),
            jax.ShapeDtypeStruct((8, _EP), jnp.int32),
            jax.ShapeDtypeStruct((8, _EP), jnp.int32),
            jax.ShapeDtypeStruct((8, _EP), jnp.float32),
        ],
    )(cnt, grk, pidm, bal)
    dest = destb.reshape(1, _N)

    xs = _sc_scatter_rows(x2, dest, _L)
    ws = _sc_scatter_rows(meta, dest, _L)

    ys = pl.pallas_call(
        _ffn_kernel,
        grid_spec=pltpu.PrefetchScalarGridSpec(
            num_scalar_prefetch=2,
            grid=(_NTC,),
            in_specs=[
                pl.BlockSpec((_TMC, _D), lambda i, s1, s2: (i, 0)),
                pl.BlockSpec((_TMC, _EP), lambda i, s1, s2: (i, 0)),
                pl.BlockSpec((_E, _D, _DF), lambda i, s1, s2: (0, 0, 0)),
                pl.BlockSpec((_E, 1, _DF), lambda i, s1, s2: (0, 0, 0)),
                pl.BlockSpec((_E, _DF, _D), lambda i, s1, s2: (0, 0, 0)),
                pl.BlockSpec((_E, 1, _D), lambda i, s1, s2: (0, 0, 0)),
            ],
            out_specs=pl.BlockSpec((_TMC, _D), lambda i, s1, s2: (i, 0)),
        ),
        out_shape=jax.ShapeDtypeStruct((_L, _D), jnp.float32),
    )(e1s, e2s, xs, ws, w1, b1.reshape(_E, 1, _DF), w2,
      b2.reshape(_E, 1, _D))

    out = _sc_gather(ys, dest)

    bal_loss = blv[0, 0]
    return out.reshape(_B, _T, _D), bal_loss


# speedup vs baseline: 1.0704x; 1.0089x over previous
import jax
import jax.numpy as jnp
from jax.experimental import pallas as pl
from jax.experimental.pallas import tpu as pltpu
from jax.experimental.pallas import tpu_sc as plsc

_B, _T, _D, _E, _DF = 64, 256, 384, 4, 1536
_N = _B * _T
_EP = 128
_TMA = 1024
_GA = _N // _TMA
_TMC = 512
_L = _N + 6 * _TMC
_NTC = _L // _TMC
_PAIR_E1 = (0, 0, 0, 1, 1, 2)
_PAIR_E2 = (1, 2, 3, 2, 3, 3)
_W = 128
_RS2 = 0.7071067811865476


def _lane_major(col, dmask):
    rows = []
    for s in range(_TMA // 128):
        v = col[s * 128:(s + 1) * 128, :]
        rows.append(jnp.sum(jnp.broadcast_to(v, (128, 128)) * dmask,
                            axis=0, keepdims=True))
    return jnp.concatenate(rows, axis=0)


def _gate_kernel(x_ref, gw_ref, gb_ref, ltri_ref,
                 meta_ref, grk_ref, pid_ref, cnt_ref, bal_ref, run_ref):
    i = pl.program_id(0)
    xt = x_ref[...]
    scores = jnp.dot(xt, gw_ref[...], preferred_element_type=jnp.float32)
    scores = scores + gb_ref[...]
    scores = jnp.nan_to_num(scores, nan=0.0)
    m = jnp.max(scores, axis=1, keepdims=True)
    ex = jnp.exp(scores - m)
    probs = ex / jnp.sum(ex, axis=1, keepdims=True)

    @pl.when(i == 0)
    def _init():
        bal_ref[...] = jnp.zeros_like(bal_ref)
        run_ref[...] = jnp.zeros_like(run_ref)
    psum = jnp.sum(probs, axis=0, keepdims=True)
    bal_ref[...] += jnp.broadcast_to(psum, bal_ref.shape)

    lane = jax.lax.broadcasted_iota(jnp.int32, probs.shape, 1)
    m1 = jnp.max(probs, axis=1, keepdims=True)
    a1 = jnp.min(jnp.where(probs == m1, lane, _EP), axis=1, keepdims=True)
    p2 = jnp.where(lane == a1, -jnp.inf, probs)
    m2 = jnp.max(p2, axis=1, keepdims=True)
    a2 = jnp.min(jnp.where(p2 == m2, lane, _EP), axis=1, keepdims=True)
    sel = (lane == a1) | (lane == a2)
    masked = jnp.where(sel, probs, 0.0)
    wgt = masked / (jnp.sum(masked, axis=1, keepdims=True) + 1e-9)
    meta_ref[...] = wgt * (lane < _E)

    emin = jnp.minimum(a1, a2)
    emax = jnp.maximum(a1, a2)
    pid = emax + jnp.where(emin == 0, -1, emin)

    onehot = (lane == pid)
    cnts = jnp.dot(ltri_ref[...], onehot.astype(jnp.bfloat16),
                   preferred_element_type=jnp.float32)
    rank = jnp.sum(jnp.where(onehot, cnts, 0.0), axis=1, keepdims=True)
    runrow = jnp.broadcast_to(run_ref[0:1, :], onehot.shape)
    grank = rank + jnp.sum(jnp.where(onehot, runrow, 0.0), axis=1,
                           keepdims=True)
    run_ref[...] += jnp.broadcast_to(
        jnp.sum(onehot.astype(jnp.float32), axis=0, keepdims=True),
        run_ref.shape)

    sub = jax.lax.broadcasted_iota(jnp.int32, (128, 128), 0)
    lan = jax.lax.broadcasted_iota(jnp.int32, (128, 128), 1)
    dmask = (sub == lan).astype(jnp.float32)
    grk_ref[...] = _lane_major(grank, dmask)
    pid_ref[...] = _lane_major(pid.astype(jnp.float32), dmask)

    @pl.when(i == _GA - 1)
    def _fin():
        cnt_ref[...] = run_ref[...]


def _dest_kernel(cnt_ref, grk_ref, pid_ref, bal_ref,
                 dest_ref, e1v_ref, e2v_ref, bl_ref):
    sub = jax.lax.broadcasted_iota(jnp.int32, (128, 128), 0)
    lan = jax.lax.broadcasted_iota(jnp.int32, (128, 128), 1)
    dmask = (sub == lan).astype(jnp.float32)

    lane1 = jax.lax.broadcasted_iota(jnp.int32, (1, _EP), 1)
    cnt = jnp.where(lane1 < 6, cnt_ref[0:1, :], 0.0)
    rup = jnp.ceil(cnt / _TMC) * _TMC
    utri = (sub <= lan).astype(jnp.float32)
    ends = jnp.dot(rup, utri, preferred_element_type=jnp.float32)
    offs = ends - rup

    grk = grk_ref[...]
    pidf = pid_ref[...]
    acc = grk
    for p in range(6):
        acc = acc + jnp.where(pidf == float(p),
                              jnp.broadcast_to(offs[:, p:p + 1], grk.shape),
                              0.0)
    dest_ref[...] = acc.astype(jnp.int32)

    ends_t = jnp.sum(jnp.broadcast_to(jnp.where(lane1 < 6, ends, 3.0e7),
                                      (128, 128)) * dmask,
                     axis=1, keepdims=True)
    ttv = (lan * _TMC).astype(jnp.float32)
    gid = jnp.sum((ttv >= ends_t).astype(jnp.float32), axis=0, keepdims=True)
    gid = jnp.minimum(gid, 5.0)
    e1 = jnp.zeros_like(gid)
    e2 = jnp.zeros_like(gid)
    for g in range(6):
        e1 = e1 + jnp.where(gid == float(g), float(_PAIR_E1[g]), 0.0)
        e2 = e2 + jnp.where(gid == float(g), float(_PAIR_E2[g]), 0.0)
    e1v_ref[...] = jnp.broadcast_to(e1, (8, _EP)).astype(jnp.int32)
    e2v_ref[...] = jnp.broadcast_to(e2, (8, _EP)).astype(jnp.int32)

    bsum = jnp.where(lane1 < _E, bal_ref[0:1, :], 0.0) * (1.0 / _N)
    bl = jnp.sum(bsum * bsum) * _E
    bl = jnp.clip(bl, 0.0, 5.0)
    bl_ref[...] = jnp.broadcast_to(bl, (8, _EP))


def _ffn_kernel(e1s_ref, e2s_ref, xs_ref, ws_ref,
                w1_ref, b1_ref, w2_ref, b2_ref, ys_ref):
    tt = pl.program_id(0)
    e1 = e1s_ref[0, tt]
    e2 = e2s_ref[0, tt]
    xt = xs_ref[...]
    ws = ws_ref[...]
    lane = jax.lax.broadcasted_iota(jnp.int32, ws.shape, 1)
    wa = jnp.sum(jnp.where(lane == e1, ws, 0.0), axis=1, keepdims=True)
    wb = jnp.sum(jnp.where(lane == e2, ws, 0.0), axis=1, keepdims=True)

    acc = jnp.zeros((_TMC, _D), dtype=jnp.float32)
    for e, w in ((e1, wa), (e2, wb)):
        t = jnp.dot(xt, w1_ref[e], preferred_element_type=jnp.float32)
        t = t + b1_ref[e]
        t = 0.5 * t * (1.0 + jax.lax.erf(t * 0.7071067811865476))
        y = jnp.dot(t, w2_ref[e], preferred_element_type=jnp.float32)
        acc = acc + w * (y + b2_ref[e])
    ys_ref[...] = acc


def _vector_mesh():
    return plsc.VectorSubcoreMesh(core_axis_name="core",
                                  subcore_axis_name="subcore")


def _sc_scatter_both(xf, meta, idx):
    @pl.kernel(out_type=[jax.ShapeDtypeStruct((_L, _D), jnp.float32),
                         jax.ShapeDtypeStruct((_L, _EP), jnp.float32)],
               mesh=_vector_mesh(), scratch_types=[])
    def skern(x_hbm, m_hbm, i_hbm, xs_hbm, ws_hbm):
        core = jax.lax.axis_index("core")

        @pl.when(core == 0)
        def _x():
            def body(s_vmem, i_vmem):
                pltpu.sync_copy(s_vmem, xs_hbm.at[i_vmem.at[0]])

            pltpu.emit_pipeline(
                body,
                grid=(_N // _W,),
                in_specs=[pl.BlockSpec((_W, _D), lambda i: (i, 0)),
                          pl.BlockSpec((1, _W), lambda i: (0, i))],
                out_specs=[],
                core_axis_name="subcore",
                dimension_semantics=(pltpu.PARALLEL,),
            )(x_hbm, i_hbm)

        @pl.when(core == 1)
        def _m():
            def body(s_vmem, i_vmem):
                pltpu.sync_copy(s_vmem, ws_hbm.at[i_vmem.at[0]])

            pltpu.emit_pipeline(
                body,
                grid=(_N // _W,),
                in_specs=[pl.BlockSpec((_W, _EP), lambda i: (i, 0)),
                          pl.BlockSpec((1, _W), lambda i: (0, i))],
                out_specs=[],
                core_axis_name="subcore",
                dimension_semantics=(pltpu.PARALLEL,),
            )(m_hbm, i_hbm)

    return skern(xf, meta, idx)


def _sc_gather(ys, idx):
    @pl.kernel(out_type=jax.ShapeDtypeStruct((_N, _D), jnp.float32),
               mesh=_vector_mesh(), scratch_types=[])
    def gkern(y_hbm, i_hbm, o_hbm):
        def body(i_vmem, o_vmem):
            pltpu.sync_copy(y_hbm.at[i_vmem.at[0]], o_vmem)

        pltpu.emit_pipeline(
            body,
            grid=(_N // _W,),
            in_specs=[pl.BlockSpec((1, _W), lambda i: (0, i))],
            out_specs=[pl.BlockSpec((_W, _D), lambda i: (i, 0))],
            core_axis_name=("core", "subcore"),
            dimension_semantics=(pltpu.PARALLEL,),
        )(i_hbm, o_hbm)

    return gkern(ys, idx)


def kernel(x, gate_w, gate_b, w1, b1, w2, b2):
    x2 = x.reshape(_N, _D)
    gw_p = jnp.zeros((_D, _EP), jnp.float32).at[:, :_E].set(gate_w)
    gb_p = jnp.full((1, _EP), -jnp.inf, jnp.float32).at[0, :_E].set(gate_b)
    ltri = jnp.tril(jnp.ones((_TMA, _TMA), jnp.bfloat16), -1)

    meta, grk, pidm, cnt, bal = pl.pallas_call(
        _gate_kernel,
        grid=(_GA,),
        in_specs=[
            pl.BlockSpec((_TMA, _D), lambda i: (i, 0)),
            pl.BlockSpec((_D, _EP), lambda i: (0, 0)),
            pl.BlockSpec((1, _EP), lambda i: (0, 0)),
            pl.BlockSpec((_TMA, _TMA), lambda i: (0, 0)),
        ],
        out_specs=[
            pl.BlockSpec((_TMA, _EP), lambda i: (i, 0)),
            pl.BlockSpec((8, _EP), lambda i: (i, 0)),
            pl.BlockSpec((8, _EP), lambda i: (i, 0)),
            pl.BlockSpec((8, _EP), lambda i: (0, 0)),
            pl.BlockSpec((8, _EP), lambda i: (0, 0)),
        ],
        out_shape=[
            jax.ShapeDtypeStruct((_N, _EP), jnp.float32),
            jax.ShapeDtypeStruct((_GA * 8, _EP), jnp.float32),
            jax.ShapeDtypeStruct((_GA * 8, _EP), jnp.float32),
            jax.ShapeDtypeStruct((8, _EP), jnp.float32),
            jax.ShapeDtypeStruct((8, _EP), jnp.float32),
        ],
        scratch_shapes=[pltpu.VMEM((8, _EP), jnp.float32)],
    )(x2, gw_p, gb_p, ltri)

    destb, e1s, e2s, blv = pl.pallas_call(
        _dest_kernel,
        grid=(1,),
        in_specs=[
            pl.BlockSpec((8, _EP), lambda i: (0, 0)),
            pl.BlockSpec((_GA * 8, _EP), lambda i: (0, 0)),
            pl.BlockSpec((_GA * 8, _EP), lambda i: (0, 0)),
            pl.BlockSpec((8, _EP), lambda i: (0, 0)),
        ],
        out_specs=[
            pl.BlockSpec((_GA * 8, _EP), lambda i: (0, 0)),
            pl.BlockSpec((8, _EP), lambda i: (0, 0)),
            pl.BlockSpec((8, _EP), lambda i: (0, 0)),
            pl.BlockSpec((8, _EP), lambda i: (0, 0)),
        ],
        out_shape=[
            jax.ShapeDtypeStruct((_GA * 8, _EP), jnp.int32),
            jax.ShapeDtypeStruct((8, _EP), jnp.int32),
            jax.ShapeDtypeStruct((8, _EP), jnp.int32),
            jax.ShapeDtypeStruct((8, _EP), jnp.float32),
        ],
    )(cnt, grk, pidm, bal)
    dest = destb.reshape(1, _N)

    xs, ws = _sc_scatter_both(x2, meta, dest)

    ys = pl.pallas_call(
        _ffn_kernel,
        grid_spec=pltpu.PrefetchScalarGridSpec(
            num_scalar_prefetch=2,
            grid=(_NTC,),
            in_specs=[
                pl.BlockSpec((_TMC, _D), lambda i, s1, s2: (i, 0)),
                pl.BlockSpec((_TMC, _EP), lambda i, s1, s2: (i, 0)),
                pl.BlockSpec((_E, _D, _DF), lambda i, s1, s2: (0, 0, 0)),
                pl.BlockSpec((_E, 1, _DF), lambda i, s1, s2: (0, 0, 0)),
                pl.BlockSpec((_E, _DF, _D), lambda i, s1, s2: (0, 0, 0)),
                pl.BlockSpec((_E, 1, _D), lambda i, s1, s2: (0, 0, 0)),
            ],
            out_specs=pl.BlockSpec((_TMC, _D), lambda i, s1, s2: (i, 0)),
        ),
        out_shape=jax.ShapeDtypeStruct((_L, _D), jnp.float32),
    )(e1s, e2s, xs, ws, w1, b1.reshape(_E, 1, _DF), w2,
      b2.reshape(_E, 1, _D))

    out = _sc_gather(ys, dest)

    bal_loss = blv[0, 0]
    return out.reshape(_B, _T, _D), bal_loss
